# Initial kernel scaffold; baseline (speedup 1.0000x reference)
import jax
import jax.numpy as jnp
from jax.experimental import pallas as pl


def kernel(pos_embedding, between_memory_index, inside_memory_index):
    raise NotImplementedError("write your pallas kernel here")


# speedup vs baseline: 3.7835x; 3.7835x over previous
import functools

import jax
import jax.numpy as jnp
from jax import lax
from jax.experimental import pallas as pl
from jax.experimental.pallas import tpu as pltpu
from jax.experimental.pallas import tpu_sc as plsc

MAXLEN = 750
D = 64
B = 4096
L = 200
NC, NS = 2, 16
NW = NC * NS

G = 128
K = 8
N_ROWS = L * B * 2
ROWS_PW = N_ROWS // NW
GROUPS_PW = ROWS_PW // G
CHUNKS = GROUPS_PW // K


def _sc_gather(table, idx_groups):
    mesh = plsc.VectorSubcoreMesh(core_axis_name="c", subcore_axis_name="s")

    @functools.partial(
        pl.kernel,
        out_type=jax.ShapeDtypeStruct((N_ROWS, D), jnp.float32),
        mesh=mesh,
        scratch_types=[
            pltpu.VMEM((K, G), jnp.int32),
            pltpu.VMEM((K * G, D), jnp.float32),
            pltpu.SemaphoreType.DMA,
        ],
        compiler_params=pltpu.CompilerParams(use_tc_tiling_on_sc=False),
    )
    def k(table_hbm, idx_hbm, out_hbm, idx_v, rows_v, gsem):
        wid = lax.axis_index("s") * NC + lax.axis_index("c")
        g_base = wid * GROUPS_PW

        def chunk(i, carry):
            g0 = g_base + i * K
            pltpu.sync_copy(idx_hbm.at[pl.ds(g0, K)], idx_v)
            copies = [
                pltpu.async_copy(
                    table_hbm.at[idx_v.at[j]],
                    rows_v.at[pl.ds(j * G, G)],
                    gsem,
                )
                for j in range(K)
            ]
            for c in copies:
                c.wait()
            pltpu.sync_copy(rows_v, out_hbm.at[pl.ds(g0 * G, K * G)])
            return carry

        lax.fori_loop(0, CHUNKS, chunk, 0)

    return k(table, idx_groups)


def kernel(pos_embedding, between_memory_index, inside_memory_index):
    idx = jnp.stack(
        (between_memory_index.T, inside_memory_index.T), axis=-1
    ).astype(jnp.int32)
    idx_groups = idx.reshape(N_ROWS // G, G)
    rows = _sc_gather(pos_embedding, idx_groups)
    return rows.reshape(L, B, 2 * D)


# speedup vs baseline: 3.8047x; 1.0056x over previous
import functools

import jax
import jax.numpy as jnp
from jax import lax
from jax.experimental import pallas as pl
from jax.experimental.pallas import tpu as pltpu
from jax.experimental.pallas import tpu_sc as plsc

MAXLEN = 750
D = 64
B = 4096
L = 200
NC, NS = 2, 16
NW = NC * NS

G = 128
K = 5
N_ROWS = L * B * 2
ROWS_PW = N_ROWS // NW
GROUPS_PW = ROWS_PW // G
CHUNKS = GROUPS_PW // K


def _sc_gather(table, idx_groups):
    mesh = plsc.VectorSubcoreMesh(core_axis_name="c", subcore_axis_name="s")

    @functools.partial(
        pl.kernel,
        out_type=jax.ShapeDtypeStruct((N_ROWS, D), jnp.float32),
        mesh=mesh,
        scratch_types=[
            pltpu.VMEM((2, K, G), jnp.int32),
            pltpu.VMEM((2, K * G, D), jnp.float32),
            pltpu.SemaphoreType.DMA,
            pltpu.SemaphoreType.DMA,
            pltpu.SemaphoreType.DMA,
        ],
        compiler_params=pltpu.CompilerParams(use_tc_tiling_on_sc=False),
    )
    def k(table_hbm, idx_hbm, out_hbm, idx_v, rows_v, gsem, isem, osem):
        wid = lax.axis_index("s") * NC + lax.axis_index("c")
        g_base = wid * GROUPS_PW

        def idx_load(c, s):
            return pltpu.async_copy(
                idx_hbm.at[pl.ds(g_base + c * K, K)], idx_v.at[s], isem
            )

        def scatter(c, s):
            return pltpu.async_copy(
                rows_v.at[s], out_hbm.at[pl.ds((g_base + c * K) * G, K * G)], osem
            )

        idx_load(0, 0)

        def chunk(i, carry):
            s = lax.rem(i, 2)

            @pl.when(i + 1 < CHUNKS)
            def _():
                idx_load(i + 1, 1 - s)

            pltpu.make_async_copy(idx_hbm.at[pl.ds(0, K)], idx_v.at[0], isem).wait()

            @pl.when(i >= 2)
            def _():
                pltpu.make_async_copy(
                    rows_v.at[0], out_hbm.at[pl.ds(0, K * G)], osem
                ).wait()

            copies = [
                pltpu.async_copy(
                    table_hbm.at[idx_v.at[s, j]],
                    rows_v.at[s, pl.ds(j * G, G)],
                    gsem,
                )
                for j in range(K)
            ]
            for c in copies:
                c.wait()
            scatter(i, s)
            return carry

        lax.fori_loop(0, CHUNKS, chunk, 0)
        for _ in range(2):
            pltpu.make_async_copy(
                rows_v.at[0], out_hbm.at[pl.ds(0, K * G)], osem
            ).wait()

    return k(table, idx_groups)


def kernel(pos_embedding, between_memory_index, inside_memory_index):
    idx = jnp.stack(
        (between_memory_index.T, inside_memory_index.T), axis=-1
    ).astype(jnp.int32)
    idx_groups = idx.reshape(N_ROWS // G, G)
    rows = _sc_gather(pos_embedding, idx_groups)
    return rows.reshape(L, B, 2 * D)


# speedup vs baseline: 5.3181x; 1.3978x over previous
import functools

import jax
import jax.numpy as jnp
from jax import lax
from jax.experimental import pallas as pl
from jax.experimental.pallas import tpu as pltpu
from jax.experimental.pallas import tpu_sc as plsc

MAXLEN = 750
D = 64
B = 4096
L = 200
NC, NS = 2, 16
NW = NC * NS

G = 128
K = 5
N_ROWS = L * B * 2
ROWS_PW = N_ROWS // NW
GROUPS_PW = ROWS_PW // G
CHUNKS = GROUPS_PW // K


def _sc_gather(table, idx_groups):
    mesh = plsc.VectorSubcoreMesh(core_axis_name="c", subcore_axis_name="s")

    @functools.partial(
        pl.kernel,
        out_type=jax.ShapeDtypeStruct((N_ROWS, D), jnp.float32),
        mesh=mesh,
        scratch_types=[
            pltpu.VMEM((2, K, G), jnp.int32),
            pltpu.VMEM((2, K * G, D), jnp.float32),
            pltpu.VMEM_SHARED((MAXLEN, D), jnp.float32),
            pltpu.SemaphoreType.DMA,
            pltpu.SemaphoreType.DMA,
            pltpu.SemaphoreType.DMA,
        ],
        compiler_params=pltpu.CompilerParams(use_tc_tiling_on_sc=False),
    )
    def k(table_hbm, idx_hbm, out_hbm, idx_v, rows_v, table_sp, gsem, isem, osem):
        wid = lax.axis_index("s") * NC + lax.axis_index("c")
        g_base = wid * GROUPS_PW

        @pl.when(lax.axis_index("s") == 0)
        def _():
            pltpu.sync_copy(table_hbm, table_sp)

        plsc.subcore_barrier()

        def idx_load(c, s):
            return pltpu.async_copy(
                idx_hbm.at[pl.ds(g_base + c * K, K)], idx_v.at[s], isem
            )

        def scatter(c, s):
            return pltpu.async_copy(
                rows_v.at[s], out_hbm.at[pl.ds((g_base + c * K) * G, K * G)], osem
            )

        idx_load(0, 0)

        def chunk(i, carry):
            s = lax.rem(i, 2)

            @pl.when(i + 1 < CHUNKS)
            def _():
                idx_load(i + 1, 1 - s)

            pltpu.make_async_copy(idx_hbm.at[pl.ds(0, K)], idx_v.at[0], isem).wait()

            @pl.when(i >= 2)
            def _():
                pltpu.make_async_copy(
                    rows_v.at[0], out_hbm.at[pl.ds(0, K * G)], osem
                ).wait()

            copies = [
                pltpu.async_copy(
                    table_sp.at[idx_v.at[s, j]],
                    rows_v.at[s, pl.ds(j * G, G)],
                    gsem,
                )
                for j in range(K)
            ]
            for c in copies:
                c.wait()
            scatter(i, s)
            return carry

        lax.fori_loop(0, CHUNKS, chunk, 0)
        for _ in range(2):
            pltpu.make_async_copy(
                rows_v.at[0], out_hbm.at[pl.ds(0, K * G)], osem
            ).wait()

    return k(table, idx_groups)


def kernel(pos_embedding, between_memory_index, inside_memory_index):
    idx = jnp.stack(
        (between_memory_index.T, inside_memory_index.T), axis=-1
    ).astype(jnp.int32)
    idx_groups = idx.reshape(N_ROWS // G, G)
    rows = _sc_gather(pos_embedding, idx_groups)
    return rows.reshape(L, B, 2 * D)


# speedup vs baseline: 35.1236x; 6.6045x over previous
import functools

import jax
import jax.numpy as jnp
from jax import lax
from jax.experimental import pallas as pl
from jax.experimental.pallas import tpu as pltpu
from jax.experimental.pallas import tpu_sc as plsc

MAXLEN = 750
D = 64
B = 4096
L = 200
NC, NS = 2, 16
NW = NC * NS

G = 128
K = 2
PAIRS = L * B
PAIRS_PW = PAIRS // NW
GROUPS_PW = PAIRS_PW // G
CHUNKS = GROUPS_PW // K


def _sc_gather(table, bet_groups, ins_groups):
    mesh = plsc.VectorSubcoreMesh(core_axis_name="c", subcore_axis_name="s")

    @functools.partial(
        pl.kernel,
        out_type=jax.ShapeDtypeStruct((PAIRS, 2 * D), jnp.float32),
        mesh=mesh,
        scratch_types=[
            pltpu.VMEM((2, 2, K, G), jnp.int32),
            pltpu.VMEM((2, 2, K * G, D), jnp.float32),
            pltpu.VMEM_SHARED((MAXLEN, D), jnp.float32),
            pltpu.SemaphoreType.DMA,
            pltpu.SemaphoreType.DMA,
            pltpu.SemaphoreType.DMA,
        ],
        compiler_params=pltpu.CompilerParams(use_tc_tiling_on_sc=False),
    )
    def k(table_hbm, bet_hbm, ins_hbm, out_hbm, idx_v, rows_v, table_sp,
          gsem, isem, osem):
        wid = lax.axis_index("s") * NC + lax.axis_index("c")
        g_base = wid * GROUPS_PW

        @pl.when(lax.axis_index("s") == 0)
        def _():
            pltpu.sync_copy(table_hbm, table_sp)

        plsc.subcore_barrier()

        def idx_load(c, s):
            pltpu.async_copy(bet_hbm.at[pl.ds(g_base + c * K, K)],
                             idx_v.at[s, 0], isem)
            pltpu.async_copy(ins_hbm.at[pl.ds(g_base + c * K, K)],
                             idx_v.at[s, 1], isem)

        def scatter(c, s):
            p0 = (g_base + c * K) * G
            pltpu.async_copy(
                rows_v.at[s, 0],
                out_hbm.at[pl.ds(p0, K * G), pl.ds(0, D)],
                osem,
            )
            pltpu.async_copy(
                rows_v.at[s, 1],
                out_hbm.at[pl.ds(p0, K * G), pl.ds(D, D)],
                osem,
            )

        def drain_scatter():
            for h in range(2):
                pltpu.make_async_copy(
                    rows_v.at[0, h],
                    out_hbm.at[pl.ds(0, K * G), pl.ds(0, D)],
                    osem,
                ).wait()

        idx_load(0, 0)

        def chunk(i, carry):
            s = lax.rem(i, 2)

            @pl.when(i + 1 < CHUNKS)
            def _():
                idx_load(i + 1, 1 - s)

            for _ in range(2):
                pltpu.make_async_copy(
                    bet_hbm.at[pl.ds(0, K)], idx_v.at[0, 0], isem
                ).wait()

            @pl.when(i >= 2)
            def _():
                drain_scatter()

            copies = []
            for h in range(2):
                for j in range(K):
                    copies.append(pltpu.async_copy(
                        table_sp.at[idx_v.at[s, h, j]],
                        rows_v.at[s, h, pl.ds(j * G, G)],
                        gsem,
                    ))
            for c in copies:
                c.wait()
            scatter(i, s)
            return carry

        lax.fori_loop(0, CHUNKS, chunk, 0)
        for _ in range(2):
            drain_scatter()

    return k(table, bet_groups, ins_groups)


def kernel(pos_embedding, between_memory_index, inside_memory_index):
    bet = between_memory_index.T.astype(jnp.int32).reshape(PAIRS // G, G)
    ins = inside_memory_index.T.astype(jnp.int32).reshape(PAIRS // G, G)
    rows = _sc_gather(pos_embedding, bet, ins)
    return rows.reshape(L, B, 2 * D)


# speedup vs baseline: 35.1433x; 1.0006x over previous
import functools

import jax
import jax.numpy as jnp
from jax import lax
from jax.experimental import pallas as pl
from jax.experimental.pallas import tpu as pltpu
from jax.experimental.pallas import tpu_sc as plsc

MAXLEN = 750
D = 64
B = 4096
L = 200
NC, NS = 2, 16
NW = NC * NS

G = 128
K = 2
PAIRS = L * B
PAIRS_PW = PAIRS // NW
GROUPS_PW = PAIRS_PW // G
CHUNKS = GROUPS_PW // K


def _sc_gather(table, bet_groups, ins_groups):
    mesh = plsc.VectorSubcoreMesh(core_axis_name="c", subcore_axis_name="s")

    @functools.partial(
        pl.kernel,
        out_type=jax.ShapeDtypeStruct((PAIRS, 2 * D), jnp.float32),
        mesh=mesh,
        scratch_types=[
            pltpu.VMEM((3, 2, K, G), jnp.int32),
            pltpu.VMEM((3, 2, K * G, D), jnp.float32),
            pltpu.VMEM_SHARED((MAXLEN, D), jnp.float32),
            pltpu.SemaphoreType.DMA,
            pltpu.SemaphoreType.DMA,
            pltpu.SemaphoreType.DMA,
        ],
        compiler_params=pltpu.CompilerParams(use_tc_tiling_on_sc=False),
    )
    def k(table_hbm, bet_hbm, ins_hbm, out_hbm, idx_v, rows_v, table_sp,
          gsem, isem, osem):
        wid = lax.axis_index("s") * NC + lax.axis_index("c")
        g_base = wid * GROUPS_PW

        @pl.when(lax.axis_index("s") == 0)
        def _():
            pltpu.sync_copy(table_hbm, table_sp)

        plsc.subcore_barrier()

        def idx_load(c, s):
            pltpu.async_copy(bet_hbm.at[pl.ds(g_base + c * K, K)],
                             idx_v.at[s, 0], isem)
            pltpu.async_copy(ins_hbm.at[pl.ds(g_base + c * K, K)],
                             idx_v.at[s, 1], isem)

        def scatter(c, s):
            p0 = (g_base + c * K) * G
            pltpu.async_copy(
                rows_v.at[s, 0],
                out_hbm.at[pl.ds(p0, K * G), pl.ds(0, D)],
                osem,
            )
            pltpu.async_copy(
                rows_v.at[s, 1],
                out_hbm.at[pl.ds(p0, K * G), pl.ds(D, D)],
                osem,
            )

        def drain_scatter():
            for h in range(2):
                pltpu.make_async_copy(
                    rows_v.at[0, h],
                    out_hbm.at[pl.ds(0, K * G), pl.ds(0, D)],
                    osem,
                ).wait()

        idx_load(0, 0)

        def chunk(i, carry):
            s = lax.rem(i, 3)

            @pl.when(i + 1 < CHUNKS)
            def _():
                idx_load(i + 1, lax.rem(i + 1, 3))

            for _ in range(2):
                pltpu.make_async_copy(
                    bet_hbm.at[pl.ds(0, K)], idx_v.at[0, 0], isem
                ).wait()

            @pl.when(i >= 3)
            def _():
                drain_scatter()

            copies = []
            for h in range(2):
                for j in range(K):
                    copies.append(pltpu.async_copy(
                        table_sp.at[idx_v.at[s, h, j]],
                        rows_v.at[s, h, pl.ds(j * G, G)],
                        gsem,
                    ))
            for c in copies:
                c.wait()
            scatter(i, s)
            return carry

        lax.fori_loop(0, CHUNKS, chunk, 0)
        for _ in range(3):
            drain_scatter()

    return k(table, bet_groups, ins_groups)


def kernel(pos_embedding, between_memory_index, inside_memory_index):
    bet = between_memory_index.T.astype(jnp.int32).reshape(PAIRS // G, G)
    ins = inside_memory_index.T.astype(jnp.int32).reshape(PAIRS // G, G)
    rows = _sc_gather(pos_embedding, bet, ins)
    return rows.reshape(L, B, 2 * D)


# speedup vs baseline: 35.1586x; 1.0004x over previous
import functools

import jax
import jax.numpy as jnp
from jax import lax
from jax.experimental import pallas as pl
from jax.experimental.pallas import tpu as pltpu
from jax.experimental.pallas import tpu_sc as plsc

MAXLEN = 750
D = 64
B = 4096
L = 200
NC, NS = 2, 16
NW = NC * NS

G = 256
K = 1
PAIRS = L * B
PAIRS_PW = PAIRS // NW
GROUPS_PW = PAIRS_PW // G
CHUNKS = GROUPS_PW // K


def _sc_gather(table, bet_groups, ins_groups):
    mesh = plsc.VectorSubcoreMesh(core_axis_name="c", subcore_axis_name="s")

    @functools.partial(
        pl.kernel,
        out_type=jax.ShapeDtypeStruct((PAIRS, 2 * D), jnp.float32),
        mesh=mesh,
        scratch_types=[
            pltpu.VMEM((3, 2, K, G), jnp.int32),
            pltpu.VMEM((3, 2, K * G, D), jnp.float32),
            pltpu.VMEM_SHARED((MAXLEN, D), jnp.float32),
            pltpu.SemaphoreType.DMA,
            pltpu.SemaphoreType.DMA,
            pltpu.SemaphoreType.DMA,
        ],
        compiler_params=pltpu.CompilerParams(use_tc_tiling_on_sc=False),
    )
    def k(table_hbm, bet_hbm, ins_hbm, out_hbm, idx_v, rows_v, table_sp,
          gsem, isem, osem):
        wid = lax.axis_index("s") * NC + lax.axis_index("c")
        g_base = wid * GROUPS_PW

        @pl.when(lax.axis_index("s") == 0)
        def _():
            pltpu.sync_copy(table_hbm, table_sp)

        plsc.subcore_barrier()

        def idx_load(c, s):
            pltpu.async_copy(bet_hbm.at[pl.ds(g_base + c * K, K)],
                             idx_v.at[s, 0], isem)
            pltpu.async_copy(ins_hbm.at[pl.ds(g_base + c * K, K)],
                             idx_v.at[s, 1], isem)

        def scatter(c, s):
            p0 = (g_base + c * K) * G
            pltpu.async_copy(
                rows_v.at[s, 0],
                out_hbm.at[pl.ds(p0, K * G), pl.ds(0, D)],
                osem,
            )
            pltpu.async_copy(
                rows_v.at[s, 1],
                out_hbm.at[pl.ds(p0, K * G), pl.ds(D, D)],
                osem,
            )

        def drain_scatter():
            for h in range(2):
                pltpu.make_async_copy(
                    rows_v.at[0, h],
                    out_hbm.at[pl.ds(0, K * G), pl.ds(0, D)],
                    osem,
                ).wait()

        idx_load(0, 0)

        def chunk(i, carry):
            s = lax.rem(i, 3)

            @pl.when(i + 1 < CHUNKS)
            def _():
                idx_load(i + 1, lax.rem(i + 1, 3))

            for _ in range(2):
                pltpu.make_async_copy(
                    bet_hbm.at[pl.ds(0, K)], idx_v.at[0, 0], isem
                ).wait()

            @pl.when(i >= 3)
            def _():
                drain_scatter()

            copies = []
            for h in range(2):
                for j in range(K):
                    copies.append(pltpu.async_copy(
                        table_sp.at[idx_v.at[s, h, j]],
                        rows_v.at[s, h, pl.ds(j * G, G)],
                        gsem,
                    ))
            for c in copies:
                c.wait()
            scatter(i, s)
            return carry

        lax.fori_loop(0, CHUNKS, chunk, 0)
        for _ in range(3):
            drain_scatter()

    return k(table, bet_groups, ins_groups)


def kernel(pos_embedding, between_memory_index, inside_memory_index):
    bet = between_memory_index.T.astype(jnp.int32).reshape(PAIRS // G, G)
    ins = inside_memory_index.T.astype(jnp.int32).reshape(PAIRS // G, G)
    rows = _sc_gather(pos_embedding, bet, ins)
    return rows.reshape(L, B, 2 * D)


# speedup vs baseline: 35.6308x; 1.0134x over previous
import functools

import jax
import jax.numpy as jnp
from jax import lax
from jax.experimental import pallas as pl
from jax.experimental.pallas import tpu as pltpu
from jax.experimental.pallas import tpu_sc as plsc

MAXLEN = 750
D = 64
B = 4096
L = 200
NC, NS = 2, 16
NW = NC * NS

G = 256
K = 1
PAIRS = L * B
PAIRS_PW = PAIRS // NW
GROUPS_PW = PAIRS_PW // G
CHUNKS = GROUPS_PW // K


def _sc_gather(table, bet_groups, ins_groups):
    mesh = plsc.VectorSubcoreMesh(core_axis_name="c", subcore_axis_name="s")

    @functools.partial(
        pl.kernel,
        out_type=jax.ShapeDtypeStruct((PAIRS, 2 * D), jnp.float32),
        mesh=mesh,
        scratch_types=[
            pltpu.VMEM((3, 2, K, G), jnp.int32),
            pltpu.VMEM((3, 2, K * G, D), jnp.float32),
            pltpu.VMEM_SHARED((MAXLEN, D), jnp.float32),
            pltpu.SemaphoreType.DMA,
            pltpu.SemaphoreType.DMA,
            pltpu.SemaphoreType.DMA,
        ],
        compiler_params=pltpu.CompilerParams(use_tc_tiling_on_sc=False),
    )
    def k(table_hbm, bet_hbm, ins_hbm, out_hbm, idx_v, rows_v, table_sp,
          gsem, isem, osem):
        wid = lax.axis_index("s") * NC + lax.axis_index("c")
        g_base = wid * GROUPS_PW

        @pl.when(lax.axis_index("s") == 0)
        def _():
            pltpu.sync_copy(table_hbm, table_sp)

        plsc.subcore_barrier()

        def idx_load(c, s):
            pltpu.async_copy(bet_hbm.at[pl.ds(g_base + c * K, K)],
                             idx_v.at[s, 0], isem)
            pltpu.async_copy(ins_hbm.at[pl.ds(g_base + c * K, K)],
                             idx_v.at[s, 1], isem)

        def drain_scatter():
            for h in range(2):
                pltpu.make_async_copy(
                    rows_v.at[0, h],
                    out_hbm.at[pl.ds(0, K * G), pl.ds(0, D)],
                    osem,
                ).wait()

        idx_load(0, 0)

        def chunk(i, carry):
            s = lax.rem(i, 3)

            @pl.when(i + 1 < CHUNKS)
            def _():
                idx_load(i + 1, lax.rem(i + 1, 3))

            for _ in range(2):
                pltpu.make_async_copy(
                    bet_hbm.at[pl.ds(0, K)], idx_v.at[0, 0], isem
                ).wait()

            @pl.when(i >= 3)
            def _():
                drain_scatter()

            copies = [
                [
                    pltpu.async_copy(
                        table_sp.at[idx_v.at[s, h, j]],
                        rows_v.at[s, h, pl.ds(j * G, G)],
                        gsem,
                    )
                    for j in range(K)
                ]
                for h in range(2)
            ]
            p0 = (g_base + i * K) * G
            for h in range(2):
                for c in copies[h]:
                    c.wait()
                pltpu.async_copy(
                    rows_v.at[s, h],
                    out_hbm.at[pl.ds(p0, K * G), pl.ds(h * D, D)],
                    osem,
                )
            return carry

        lax.fori_loop(0, CHUNKS, chunk, 0)
        for _ in range(3):
            drain_scatter()

    return k(table, bet_groups, ins_groups)


def kernel(pos_embedding, between_memory_index, inside_memory_index):
    bet = between_memory_index.T.astype(jnp.int32).reshape(PAIRS // G, G)
    ins = inside_memory_index.T.astype(jnp.int32).reshape(PAIRS // G, G)
    rows = _sc_gather(pos_embedding, bet, ins)
    return rows.reshape(L, B, 2 * D)


# speedup vs baseline: 36.1635x; 1.0150x over previous
import functools

import jax
import jax.numpy as jnp
from jax import lax
from jax.experimental import pallas as pl
from jax.experimental.pallas import tpu as pltpu
from jax.experimental.pallas import tpu_sc as plsc

MAXLEN = 750
D = 64
B = 4096
L = 200
NC, NS = 2, 16
NW = NC * NS

G = 256
K = 1
PAIRS = L * B
PAIRS_PW = PAIRS // NW
GROUPS_PW = PAIRS_PW // G
CHUNKS = GROUPS_PW // K


def _sc_gather(table, bet_groups, ins_groups):
    mesh = plsc.VectorSubcoreMesh(core_axis_name="c", subcore_axis_name="s")

    @functools.partial(
        pl.kernel,
        out_type=jax.ShapeDtypeStruct((PAIRS, 2 * D), jnp.float32),
        mesh=mesh,
        scratch_types=[
            pltpu.VMEM((3, 2, K, G), jnp.int32),
            pltpu.VMEM((3, 2, K * G, D), jnp.float32),
            pltpu.VMEM_SHARED((MAXLEN, D), jnp.float32),
            pltpu.SemaphoreType.DMA,
            pltpu.SemaphoreType.DMA,
            pltpu.SemaphoreType.DMA,
        ],
        compiler_params=pltpu.CompilerParams(use_tc_tiling_on_sc=False),
    )
    def k(table_hbm, bet_hbm, ins_hbm, out_hbm, idx_v, rows_v, table_sp,
          gsem, isem, osem):
        wid = lax.axis_index("s") * NC + lax.axis_index("c")
        g_base = wid * GROUPS_PW

        @pl.when(lax.axis_index("s") == 0)
        def _():
            pltpu.sync_copy(table_hbm, table_sp)

        plsc.subcore_barrier()

        def idx_load(c, s):
            pltpu.async_copy(bet_hbm.at[pl.ds(g_base + c * K, K)],
                             idx_v.at[s, 0], isem)
            pltpu.async_copy(ins_hbm.at[pl.ds(g_base + c * K, K)],
                             idx_v.at[s, 1], isem)

        def drain_scatter():
            for h in range(2):
                pltpu.make_async_copy(
                    rows_v.at[0, h],
                    out_hbm.at[pl.ds(0, K * G), pl.ds(0, D)],
                    osem,
                ).wait()

        def fire_gathers(c, s):
            for h in range(2):
                for j in range(K):
                    pltpu.async_copy(
                        table_sp.at[idx_v.at[s, h, j]],
                        rows_v.at[s, h, pl.ds(j * G, G)],
                        gsem,
                    )

        def drain_idx_pair():
            for _ in range(2):
                pltpu.make_async_copy(
                    bet_hbm.at[pl.ds(0, K)], idx_v.at[0, 0], isem
                ).wait()

        def drain_gathers():
            for h in range(2):
                for _ in range(K):
                    pltpu.make_async_copy(
                        out_hbm.at[pl.ds(0, G), pl.ds(0, D)],
                        rows_v.at[0, 0, pl.ds(0, G)],
                        gsem,
                    ).wait()

        idx_load(0, 0)
        idx_load(1, 1)
        drain_idx_pair()
        fire_gathers(0, 0)

        def chunk(i, carry):
            s = lax.rem(i, 3)
            s1 = lax.rem(i + 1, 3)

            @pl.when(i + 2 < CHUNKS)
            def _():
                idx_load(i + 2, lax.rem(i + 2, 3))

            @pl.when(i + 1 < CHUNKS)
            def _():
                drain_idx_pair()

                @pl.when(i >= 2)
                def _():
                    drain_scatter()

                fire_gathers(i + 1, s1)

            drain_gathers()
            p0 = (g_base + i * K) * G
            for h in range(2):
                pltpu.async_copy(
                    rows_v.at[s, h],
                    out_hbm.at[pl.ds(p0, K * G), pl.ds(h * D, D)],
                    osem,
                )
            return carry

        lax.fori_loop(0, CHUNKS, chunk, 0)
        for _ in range(3):
            drain_scatter()

    return k(table, bet_groups, ins_groups)


def kernel(pos_embedding, between_memory_index, inside_memory_index):
    bet = between_memory_index.T.astype(jnp.int32).reshape(PAIRS // G, G)
    ins = inside_memory_index.T.astype(jnp.int32).reshape(PAIRS // G, G)
    rows = _sc_gather(pos_embedding, bet, ins)
    return rows.reshape(L, B, 2 * D)
